# Initial kernel scaffold; baseline (speedup 1.0000x reference)
import jax
import jax.numpy as jnp
from jax.experimental import pallas as pl


def kernel(edge_attr, W0, W1, W2):
    raise NotImplementedError("write your pallas kernel here")


# speedup vs baseline: 1.4732x; 1.4732x over previous
import functools

import jax
import jax.numpy as jnp
from jax import lax
from jax.experimental import pallas as pl
from jax.experimental.pallas import tpu as pltpu
from jax.experimental.pallas import tpu_sc as plsc

D = 32
D0, D1, D2 = 22, 6, 2
R = D0 * D1 * D2
NC, NS = 2, 16
NW = NC * NS
CHUNK = 400


def _sc_body(n_per_w, n_chunks,
             attr_hbm, w0_hbm, w1_hbm, w2_hbm, out_hbm,
             w0_v, w1_v, w2_v, t_v, attr_v, out_v):
    wid = lax.axis_index("s") * NC + lax.axis_index("c")

    pltpu.sync_copy(w0_hbm, w0_v)
    pltpu.sync_copy(w1_hbm, w1_v)
    pltpu.sync_copy(w2_hbm, w2_v)

    def build_row(c, carry):
        i = c // (D1 * D2)
        r = c - i * (D1 * D2)
        j = r // D2
        k = r - j * D2
        for h in (0, 16):
            t = (w0_v[pl.ds(i * D + h, 16)]
                 + w1_v[pl.ds(j * D + h, 16)]
                 + w2_v[pl.ds(k * D + h, 16)])
            t_v[pl.ds(c * D + h, 16)] = t
        return carry

    lax.fori_loop(0, R, build_row, 0)

    lane = lax.broadcasted_iota(jnp.int32, (16,), 0)
    iota3 = lane * 3
    oiota = lane * D
    abase = wid * (n_per_w * 3)
    obase = wid * (n_per_w * D)

    def chunk_body(ci, carry):
        pltpu.sync_copy(attr_hbm.at[pl.ds(abase + ci * (CHUNK * 3), CHUNK * 3)],
                        attr_v)
        for g in range(CHUNK // 16):
            idx0 = iota3 + (g * 48)
            a0 = plsc.load_gather(attr_v, [idx0])
            a1 = plsc.load_gather(attr_v, [idx0 + 1])
            a2 = plsc.load_gather(attr_v, [idx0 + 2])
            rowbase = a0 * (D1 * D2 * D) + a1 * (D2 * D) + a2 * D
            ob = oiota + (g * 16 * D)
            for d in range(D):
                v = plsc.load_gather(t_v, [rowbase + d])
                plsc.store_scatter(out_v, [ob + d], v)
        pltpu.sync_copy(out_v,
                        out_hbm.at[pl.ds(obase + ci * (CHUNK * D), CHUNK * D)])
        return carry

    lax.fori_loop(0, n_chunks, chunk_body, 0)


def kernel(edge_attr, W0, W1, W2):
    E = edge_attr.shape[0]
    assert E % (NW * CHUNK) == 0
    n_per_w = E // NW
    n_chunks = n_per_w // CHUNK

    attr = jnp.asarray(edge_attr, jnp.int32).reshape(-1)
    w0 = W0.reshape(-1)
    w1 = W1.reshape(-1)
    w2 = W2.reshape(-1)

    mesh = plsc.VectorSubcoreMesh(core_axis_name="c", subcore_axis_name="s")
    body = functools.partial(_sc_body, n_per_w, n_chunks)
    out_flat = pl.kernel(
        body,
        out_type=jax.ShapeDtypeStruct((E * D,), jnp.float32),
        mesh=mesh,
        compiler_params=pltpu.CompilerParams(needs_layout_passes=False),
        scratch_types=[
            pltpu.VMEM((D0 * D,), jnp.float32),
            pltpu.VMEM((D1 * D,), jnp.float32),
            pltpu.VMEM((D2 * D,), jnp.float32),
            pltpu.VMEM((R * D,), jnp.float32),
            pltpu.VMEM((CHUNK * 3,), jnp.int32),
            pltpu.VMEM((CHUNK * D,), jnp.float32),
        ],
    )(attr, w0, w1, w2)
    return out_flat.reshape(E, D)


# speedup vs baseline: 1.5890x; 1.0786x over previous
import functools

import jax
import jax.numpy as jnp
from jax import lax
from jax.experimental import pallas as pl
from jax.experimental.pallas import tpu as pltpu
from jax.experimental.pallas import tpu_sc as plsc

D = 32
D0, D1, D2 = 22, 6, 2
R = D0 * D1 * D2
NC, NS = 2, 16
NW = NC * NS
CHUNK = 2000


def _sc_body(n_per_w, n_chunks,
             attr_hbm, w0_hbm, w1_hbm, w2_hbm, out_hbm,
             w0_v, w1_v, w2_v, t_v, attr_v, out_v):
    wid = lax.axis_index("s") * NC + lax.axis_index("c")

    pltpu.sync_copy(w0_hbm, w0_v)
    pltpu.sync_copy(w1_hbm, w1_v)
    pltpu.sync_copy(w2_hbm, w2_v)

    @plsc.parallel_loop(0, R, step=1, unroll=4)
    def build_row(c):
        i = c // (D1 * D2)
        r = c - i * (D1 * D2)
        j = r // D2
        k = r - j * D2
        for h in (0, 16):
            t = (w0_v[pl.ds(i * D + h, 16)]
                 + w1_v[pl.ds(j * D + h, 16)]
                 + w2_v[pl.ds(k * D + h, 16)])
            t_v[pl.ds(c * D + h, 16)] = t

    lane = lax.broadcasted_iota(jnp.int32, (16,), 0)
    iota3 = lane * 3
    oiota = lane * D
    abase = wid * (n_per_w * 3)
    obase = wid * (n_per_w * D)

    def chunk_body(ci, carry):
        pltpu.sync_copy(attr_hbm.at[pl.ds(abase + ci * (CHUNK * 3), CHUNK * 3)],
                        attr_v)

        @plsc.parallel_loop(0, CHUNK // 16, step=1, unroll=4)
        def group_body(g):
            idx0 = iota3 + g * 48
            a0 = plsc.load_gather(attr_v, [idx0])
            a1 = plsc.load_gather(attr_v, [idx0 + 1])
            a2 = plsc.load_gather(attr_v, [idx0 + 2])
            rowbase = a0 * (D1 * D2 * D) + a1 * (D2 * D) + a2 * D
            ob = oiota + g * (16 * D)
            for d in range(D):
                v = plsc.load_gather(t_v, [rowbase + d])
                plsc.store_scatter(out_v, [ob + d], v)

        pltpu.sync_copy(out_v,
                        out_hbm.at[pl.ds(obase + ci * (CHUNK * D), CHUNK * D)])
        return carry

    lax.fori_loop(0, n_chunks, chunk_body, 0)


def kernel(edge_attr, W0, W1, W2):
    E = edge_attr.shape[0]
    assert E % (NW * CHUNK) == 0
    n_per_w = E // NW
    n_chunks = n_per_w // CHUNK

    attr = jnp.asarray(edge_attr, jnp.int32).reshape(-1)
    w0 = W0.reshape(-1)
    w1 = W1.reshape(-1)
    w2 = W2.reshape(-1)

    mesh = plsc.VectorSubcoreMesh(core_axis_name="c", subcore_axis_name="s")
    body = functools.partial(_sc_body, n_per_w, n_chunks)
    out_flat = pl.kernel(
        body,
        out_type=jax.ShapeDtypeStruct((E * D,), jnp.float32),
        mesh=mesh,
        compiler_params=pltpu.CompilerParams(needs_layout_passes=False),
        scratch_types=[
            pltpu.VMEM((D0 * D,), jnp.float32),
            pltpu.VMEM((D1 * D,), jnp.float32),
            pltpu.VMEM((D2 * D,), jnp.float32),
            pltpu.VMEM((R * D,), jnp.float32),
            pltpu.VMEM((CHUNK * 3,), jnp.int32),
            pltpu.VMEM((CHUNK * D,), jnp.float32),
        ],
    )(attr, w0, w1, w2)
    return out_flat.reshape(E, D)


# speedup vs baseline: 1.9291x; 1.2140x over previous
import functools

import jax
import jax.numpy as jnp
from jax import lax
from jax.experimental import pallas as pl
from jax.experimental.pallas import tpu as pltpu
from jax.experimental.pallas import tpu_sc as plsc

D = 32
D0, D1, D2 = 22, 6, 2
R = D0 * D1 * D2
NC, NS = 2, 16
NW = NC * NS
CHUNK = 1280
CB = CHUNK // 128


def _sc_body(n_chunks, n_iters,
             attr_hbm, w0_hbm, w1_hbm, w2_hbm, out_hbm,
             w0_v, w1_v, w2_v, t_v, attr_v, out_v):
    wid = lax.axis_index("s") * NC + lax.axis_index("c")

    pltpu.sync_copy(w0_hbm, w0_v)
    pltpu.sync_copy(w1_hbm, w1_v)
    pltpu.sync_copy(w2_hbm, w2_v)

    lane = lax.broadcasted_iota(jnp.int32, (16,), 0)
    iota3 = lane * 3

    @plsc.parallel_loop(0, R, step=1, unroll=4)
    def build_row(c):
        i = c // (D1 * D2)
        r = c - i * (D1 * D2)
        j = r // D2
        k = r - j * D2
        for h in (0, 16):
            t = (w0_v[pl.ds(i * D + h, 16)]
                 + w1_v[pl.ds(j * D + h, 16)]
                 + w2_v[pl.ds(k * D + h, 16)])
            t_v[pl.ds(c * D + h, 16)] = t

    def chunk_body(ci, carry):
        c = wid + ci * NW

        @pl.when(c < n_chunks)
        def _():
            pltpu.sync_copy(attr_hbm.at[pl.ds(c * (CHUNK * 3), CHUNK * 3)],
                            attr_v)

            @plsc.parallel_loop(0, CHUNK // 16, step=1, unroll=4)
            def group_body(g):
                idx0 = iota3 + g * 48
                a0 = plsc.load_gather(attr_v, [idx0])
                a1 = plsc.load_gather(attr_v, [idx0 + 1])
                a2 = plsc.load_gather(attr_v, [idx0 + 2])
                rowbase = a0 * (D1 * D2 * D) + a1 * (D2 * D) + a2 * D
                leb = g // 8
                le0 = (g % 8) * 16
                for d in range(D):
                    v = plsc.load_gather(t_v, [rowbase + d])
                    out_v[d // 8, leb, d % 8, pl.ds(le0, 16)] = v

            pltpu.sync_copy(out_v, out_hbm.at[:, pl.ds(c * CB, CB)])

        return carry

    lax.fori_loop(0, n_iters, chunk_body, 0)


def kernel(edge_attr, W0, W1, W2):
    E = edge_attr.shape[0]
    assert E % CHUNK == 0
    n_chunks = E // CHUNK
    n_iters = (n_chunks + NW - 1) // NW
    EB = E // 128

    attr = jnp.asarray(edge_attr, jnp.int32).reshape(-1)

    mesh = plsc.VectorSubcoreMesh(core_axis_name="c", subcore_axis_name="s")
    body = functools.partial(_sc_body, n_chunks, n_iters)
    out4 = pl.kernel(
        body,
        out_type=jax.ShapeDtypeStruct((4, EB, 8, 128), jnp.float32),
        mesh=mesh,
        compiler_params=pltpu.CompilerParams(needs_layout_passes=False),
        scratch_types=[
            pltpu.VMEM((D0 * D,), jnp.float32),
            pltpu.VMEM((D1 * D,), jnp.float32),
            pltpu.VMEM((D2 * D,), jnp.float32),
            pltpu.VMEM((R * D,), jnp.float32),
            pltpu.VMEM((CHUNK * 3,), jnp.int32),
            pltpu.VMEM((4, CB, 8, 128), jnp.float32),
        ],
    )(attr, W0.reshape(-1), W1.reshape(-1), W2.reshape(-1))
    return out4.transpose(1, 3, 0, 2).reshape(E, D)


# speedup vs baseline: 11.3718x; 5.8950x over previous
import functools

import jax
import jax.numpy as jnp
from jax import lax
from jax.experimental import pallas as pl
from jax.experimental.pallas import tpu as pltpu
from jax.experimental.pallas import tpu_sc as plsc

D = 32
D0, D1, D2 = 22, 6, 2
R = D0 * D1 * D2
NC, NS = 2, 16
NW = NC * NS
CHUNK = 1280
CB = CHUNK // 128


def _sc_body(n_chunks, n_iters,
             attr_hbm, w0_hbm, w1_hbm, w2_hbm, out_hbm,
             w0_v, w1_v, w2_v, t_v, attr_v, out_v):
    wid = lax.axis_index("s") * NC + lax.axis_index("c")

    pltpu.sync_copy(w0_hbm, w0_v)
    pltpu.sync_copy(w1_hbm, w1_v)
    pltpu.sync_copy(w2_hbm, w2_v)

    @plsc.parallel_loop(0, R, step=1, unroll=4)
    def build_row(c):
        i = c // (D1 * D2)
        r = c - i * (D1 * D2)
        j = r // D2
        k = r - j * D2
        for h in (0, 16):
            t = (w0_v[pl.ds(i * D + h, 16)]
                 + w1_v[pl.ds(j * D + h, 16)]
                 + w2_v[pl.ds(k * D + h, 16)])
            t_v[pl.ds(c * D + h, 16)] = t

    def chunk_body(ci, carry):
        c = wid + ci * NW

        @pl.when(c < n_chunks)
        def _():
            pltpu.sync_copy(attr_hbm.at[pl.ds(c * (CB * 512), CB * 512)],
                            attr_v)

            @plsc.parallel_loop(0, CHUNK // 16, step=1, unroll=4)
            def group_body(g):
                a_off = (g // 8) * 512 + (g % 8) * 16
                a0 = attr_v[pl.ds(a_off, 16)]
                a1 = attr_v[pl.ds(a_off + 128, 16)]
                a2 = attr_v[pl.ds(a_off + 256, 16)]
                rowbase = a0 * (D1 * D2 * D) + a1 * (D2 * D) + a2 * D
                leb = g // 8
                le0 = (g % 8) * 16
                for d in range(D):
                    v = plsc.load_gather(t_v, [rowbase + d])
                    out_v[d // 8, leb, d % 8, pl.ds(le0, 16)] = v

            pltpu.sync_copy(out_v, out_hbm.at[:, pl.ds(c * CB, CB)])

        return carry

    lax.fori_loop(0, n_iters, chunk_body, 0)


def kernel(edge_attr, W0, W1, W2):
    E = edge_attr.shape[0]
    assert E % CHUNK == 0
    n_chunks = E // CHUNK
    n_iters = (n_chunks + NW - 1) // NW
    EB = E // 128

    attr = jnp.asarray(edge_attr, jnp.int32)
    attr = attr.T.reshape(3, EB, 128).transpose(1, 0, 2)
    attr = jnp.pad(attr, ((0, 0), (0, 1), (0, 0))).reshape(-1)

    mesh = plsc.VectorSubcoreMesh(core_axis_name="c", subcore_axis_name="s")
    body = functools.partial(_sc_body, n_chunks, n_iters)
    out4 = pl.kernel(
        body,
        out_type=jax.ShapeDtypeStruct((4, EB, 8, 128), jnp.float32),
        mesh=mesh,
        compiler_params=pltpu.CompilerParams(needs_layout_passes=False),
        scratch_types=[
            pltpu.VMEM((D0 * D,), jnp.float32),
            pltpu.VMEM((D1 * D,), jnp.float32),
            pltpu.VMEM((D2 * D,), jnp.float32),
            pltpu.VMEM((R * D,), jnp.float32),
            pltpu.VMEM((CB * 512,), jnp.int32),
            pltpu.VMEM((4, CB, 8, 128), jnp.float32),
        ],
    )(attr, W0.reshape(-1), W1.reshape(-1), W2.reshape(-1))
    return out4.transpose(1, 3, 0, 2).reshape(E, D)


# speedup vs baseline: 12.4799x; 1.0974x over previous
import functools

import jax
import jax.numpy as jnp
from jax import lax
from jax.experimental import pallas as pl
from jax.experimental.pallas import tpu as pltpu
from jax.experimental.pallas import tpu_sc as plsc

D = 32
D0, D1, D2 = 22, 6, 2
R = D0 * D1 * D2
NC, NS = 2, 16
NW = NC * NS
CHUNK = 1280
CB = CHUNK // 128


def _sc_body(n_chunks, n_super,
             attr_hbm, w0_hbm, w1_hbm, w2_hbm, out_hbm,
             w0_v, w1_v, w2_v, t_v,
             attr_va, attr_vb, out_va, out_vb,
             in_sa, in_sb, out_sa, out_sb):
    wid = lax.axis_index("s") * NC + lax.axis_index("c")

    pltpu.sync_copy(w0_hbm, w0_v)
    pltpu.sync_copy(w1_hbm, w1_v)
    pltpu.sync_copy(w2_hbm, w2_v)

    @plsc.parallel_loop(0, R, step=1, unroll=4)
    def build_row(c):
        i = c // (D1 * D2)
        r = c - i * (D1 * D2)
        j = r // D2
        k = r - j * D2
        for h in (0, 16):
            t = (w0_v[pl.ds(i * D + h, 16)]
                 + w1_v[pl.ds(j * D + h, 16)]
                 + w2_v[pl.ds(k * D + h, 16)])
            t_v[pl.ds(c * D + h, 16)] = t

    def in_copy(c, attr_v, sem):
        return pltpu.make_async_copy(
            attr_hbm.at[pl.ds(c * (CB * 512), CB * 512)], attr_v, sem)

    def out_copy(c, out_v, sem):
        return pltpu.make_async_copy(
            out_v, out_hbm.at[:, pl.ds(c * CB, CB)], sem)

    def compute(attr_v, out_v):
        @plsc.parallel_loop(0, CHUNK // 16, step=1, unroll=4)
        def group_body(g):
            a_off = (g // 8) * 512 + (g % 8) * 16
            a0 = attr_v[pl.ds(a_off, 16)]
            a1 = attr_v[pl.ds(a_off + 128, 16)]
            a2 = attr_v[pl.ds(a_off + 256, 16)]
            rowbase = a0 * (D1 * D2 * D) + a1 * (D2 * D) + a2 * D
            leb = g // 8
            le0 = (g % 8) * 16
            for d in range(D):
                v = plsc.load_gather(t_v, [rowbase + d])
                out_v[d // 8, leb, d % 8, pl.ds(le0, 16)] = v

    in_copy(wid, attr_va, in_sa).start()

    def super_body(j, carry):
        c0 = wid + (2 * j) * NW

        def stage(c, attr_v, in_sem, out_v, out_sem,
                  c_next, attr_nv, in_nsem):
            @pl.when(c < n_chunks)
            def _():
                in_copy(c, attr_v, in_sem).wait()

                @pl.when(c_next < n_chunks)
                def _():
                    in_copy(c_next, attr_nv, in_nsem).start()

                @pl.when(c >= 2 * NW)
                def _():
                    out_copy(c - 2 * NW, out_v, out_sem).wait()

                compute(attr_v, out_v)
                out_copy(c, out_v, out_sem).start()

        stage(c0, attr_va, in_sa, out_va, out_sa, c0 + NW, attr_vb, in_sb)
        stage(c0 + NW, attr_vb, in_sb, out_vb, out_sb,
              c0 + 2 * NW, attr_va, in_sa)
        return carry

    lax.fori_loop(0, n_super, super_body, 0)

    last_a = ((n_chunks - 1 - wid) // (2 * NW)) * (2 * NW) + wid

    @pl.when(wid < n_chunks)
    def _():
        out_copy(last_a, out_va, out_sa).wait()

    last_b = ((n_chunks - 1 - wid - NW) // (2 * NW)) * (2 * NW) + wid + NW

    @pl.when(wid + NW < n_chunks)
    def _():
        out_copy(last_b, out_vb, out_sb).wait()


def kernel(edge_attr, W0, W1, W2):
    E = edge_attr.shape[0]
    assert E % CHUNK == 0
    n_chunks = E // CHUNK
    n_iters = (n_chunks + NW - 1) // NW
    n_super = (n_iters + 1) // 2
    EB = E // 128

    attr = jnp.asarray(edge_attr, jnp.int32)
    attr = attr.T.reshape(3, EB, 128).transpose(1, 0, 2)
    attr = jnp.pad(attr, ((0, 0), (0, 1), (0, 0))).reshape(-1)

    mesh = plsc.VectorSubcoreMesh(core_axis_name="c", subcore_axis_name="s")
    body = functools.partial(_sc_body, n_chunks, n_super)
    out4 = pl.kernel(
        body,
        out_type=jax.ShapeDtypeStruct((4, EB, 8, 128), jnp.float32),
        mesh=mesh,
        compiler_params=pltpu.CompilerParams(needs_layout_passes=False),
        scratch_types=[
            pltpu.VMEM((D0 * D,), jnp.float32),
            pltpu.VMEM((D1 * D,), jnp.float32),
            pltpu.VMEM((D2 * D,), jnp.float32),
            pltpu.VMEM((R * D,), jnp.float32),
            pltpu.VMEM((CB * 512,), jnp.int32),
            pltpu.VMEM((CB * 512,), jnp.int32),
            pltpu.VMEM((4, CB, 8, 128), jnp.float32),
            pltpu.VMEM((4, CB, 8, 128), jnp.float32),
            pltpu.SemaphoreType.DMA,
            pltpu.SemaphoreType.DMA,
            pltpu.SemaphoreType.DMA,
            pltpu.SemaphoreType.DMA,
        ],
    )(attr, W0.reshape(-1), W1.reshape(-1), W2.reshape(-1))
    return out4.transpose(1, 3, 0, 2).reshape(E, D)


# speedup vs baseline: 54.4370x; 4.3620x over previous
import functools

import jax
import jax.numpy as jnp
from jax import lax
from jax.experimental import pallas as pl
from jax.experimental.pallas import tpu as pltpu
from jax.experimental.pallas import tpu_sc as plsc

D = 32
D0, D1, D2 = 22, 6, 2
R = D0 * D1 * D2
NC, NS = 2, 16
NW = NC * NS
TS = D + 1
CHUNK = 1280
CB = CHUNK // 128


def _sc_body(n_chunks, n_super,
             attr_hbm, w0_hbm, w1_hbm, w2_hbm, out_hbm,
             w0_v, w1_v, w2_v, t_v,
             attr_va, attr_vb, out_va, out_vb,
             in_sa, in_sb, out_sa, out_sb):
    wid = lax.axis_index("s") * NC + lax.axis_index("c")

    pltpu.sync_copy(w0_hbm, w0_v)
    pltpu.sync_copy(w1_hbm, w1_v)
    pltpu.sync_copy(w2_hbm, w2_v)

    lane = lax.broadcasted_iota(jnp.int32, (16,), 0)

    @plsc.parallel_loop(0, R, step=1, unroll=4)
    def build_row(c):
        i = c // (D1 * D2)
        r = c - i * (D1 * D2)
        j = r // D2
        k = r - j * D2
        for h in (0, 16):
            t = (w0_v[pl.ds(i * D + h, 16)]
                 + w1_v[pl.ds(j * D + h, 16)]
                 + w2_v[pl.ds(k * D + h, 16)])
            plsc.store_scatter(t_v, [lane + (c * TS + h)], t)

    def in_copy(c, attr_v, sem):
        return pltpu.make_async_copy(
            attr_hbm.at[pl.ds(c * (CB * 512), CB * 512)], attr_v, sem)

    def out_copy(c, out_v, sem):
        return pltpu.make_async_copy(
            out_v, out_hbm.at[:, pl.ds(c * CB, CB)], sem)

    def compute(attr_v, out_v):
        @plsc.parallel_loop(0, CHUNK // 16, step=1, unroll=4)
        def group_body(g):
            a_off = (g // 8) * 512 + (g % 8) * 16
            a0 = attr_v[pl.ds(a_off, 16)]
            a1 = attr_v[pl.ds(a_off + 128, 16)]
            a2 = attr_v[pl.ds(a_off + 256, 16)]
            rowbase = a0 * (D1 * D2 * TS) + a1 * (D2 * TS) + a2 * TS
            leb = g // 8
            le0 = (g % 8) * 16
            for d in range(D):
                v = plsc.load_gather(t_v, [rowbase + d])
                out_v[d // 8, leb, d % 8, pl.ds(le0, 16)] = v

    in_copy(wid, attr_va, in_sa).start()

    def super_body(j, carry):
        c0 = wid + (2 * j) * NW

        def stage(c, attr_v, in_sem, out_v, out_sem,
                  c_next, attr_nv, in_nsem):
            @pl.when(c < n_chunks)
            def _():
                in_copy(c, attr_v, in_sem).wait()

                @pl.when(c_next < n_chunks)
                def _():
                    in_copy(c_next, attr_nv, in_nsem).start()

                @pl.when(c >= 2 * NW)
                def _():
                    out_copy(c - 2 * NW, out_v, out_sem).wait()

                compute(attr_v, out_v)
                out_copy(c, out_v, out_sem).start()

        stage(c0, attr_va, in_sa, out_va, out_sa, c0 + NW, attr_vb, in_sb)
        stage(c0 + NW, attr_vb, in_sb, out_vb, out_sb,
              c0 + 2 * NW, attr_va, in_sa)
        return carry

    lax.fori_loop(0, n_super, super_body, 0)

    last_a = ((n_chunks - 1 - wid) // (2 * NW)) * (2 * NW) + wid

    @pl.when(wid < n_chunks)
    def _():
        out_copy(last_a, out_va, out_sa).wait()

    last_b = ((n_chunks - 1 - wid - NW) // (2 * NW)) * (2 * NW) + wid + NW

    @pl.when(wid + NW < n_chunks)
    def _():
        out_copy(last_b, out_vb, out_sb).wait()


def kernel(edge_attr, W0, W1, W2):
    E = edge_attr.shape[0]
    assert E % CHUNK == 0
    n_chunks = E // CHUNK
    n_iters = (n_chunks + NW - 1) // NW
    n_super = (n_iters + 1) // 2
    EB = E // 128

    attr = jnp.asarray(edge_attr, jnp.int32)
    attr = attr.T.reshape(3, EB, 128).transpose(1, 0, 2)
    attr = jnp.pad(attr, ((0, 0), (0, 1), (0, 0))).reshape(-1)

    mesh = plsc.VectorSubcoreMesh(core_axis_name="c", subcore_axis_name="s")
    body = functools.partial(_sc_body, n_chunks, n_super)
    out4 = pl.kernel(
        body,
        out_type=jax.ShapeDtypeStruct((4, EB, 8, 128), jnp.float32),
        mesh=mesh,
        compiler_params=pltpu.CompilerParams(needs_layout_passes=False),
        scratch_types=[
            pltpu.VMEM((D0 * D,), jnp.float32),
            pltpu.VMEM((D1 * D,), jnp.float32),
            pltpu.VMEM((D2 * D,), jnp.float32),
            pltpu.VMEM((R * TS,), jnp.float32),
            pltpu.VMEM((CB * 512,), jnp.int32),
            pltpu.VMEM((CB * 512,), jnp.int32),
            pltpu.VMEM((4, CB, 8, 128), jnp.float32),
            pltpu.VMEM((4, CB, 8, 128), jnp.float32),
            pltpu.SemaphoreType.DMA,
            pltpu.SemaphoreType.DMA,
            pltpu.SemaphoreType.DMA,
            pltpu.SemaphoreType.DMA,
        ],
    )(attr, W0.reshape(-1), W1.reshape(-1), W2.reshape(-1))
    return out4.transpose(1, 3, 0, 2).reshape(E, D)


# speedup vs baseline: 72.8911x; 1.3390x over previous
import functools

import jax
import jax.numpy as jnp
from jax import lax
from jax.experimental import pallas as pl
from jax.experimental.pallas import tpu as pltpu
from jax.experimental.pallas import tpu_sc as plsc

D = 32
D0, D1, D2 = 22, 6, 2
R = D0 * D1 * D2
NC, NS = 2, 16
NW = NC * NS
TS = D + 1
CHUNK = 1280
CB = CHUNK // 128


def _sc_body(n_chunks, n_super,
             attr_hbm, w0_hbm, w1_hbm, w2_hbm, out_hbm,
             w0_v, w1_v, w2_v, t_v,
             attr_va, attr_vb, out_va, out_vb,
             in_sa, in_sb, out_sa, out_sb):
    wid = lax.axis_index("s") * NC + lax.axis_index("c")

    pltpu.sync_copy(w0_hbm, w0_v)
    pltpu.sync_copy(w1_hbm, w1_v)
    pltpu.sync_copy(w2_hbm, w2_v)

    lane = lax.broadcasted_iota(jnp.int32, (16,), 0)

    @plsc.parallel_loop(0, R, step=1, unroll=4)
    def build_row(c):
        i = c // (D1 * D2)
        r = c - i * (D1 * D2)
        j = r // D2
        k = r - j * D2
        for h in (0, 16):
            t = (w0_v[pl.ds(i * D + h, 16)]
                 + w1_v[pl.ds(j * D + h, 16)]
                 + w2_v[pl.ds(k * D + h, 16)])
            plsc.store_scatter(t_v, [lane + (c * TS + h)], t)

    def in_copy(c, attr_v, sem):
        return pltpu.make_async_copy(
            attr_hbm.at[pl.ds(c * (CB * 512), CB * 512)], attr_v, sem)

    def out_copy(c, out_v, sem):
        return pltpu.make_async_copy(
            out_v, out_hbm.at[:, pl.ds(c * CB, CB)], sem)

    def compute(attr_v, out_v):
        @plsc.parallel_loop(0, CHUNK // 16, step=1, unroll=8)
        def group_body(g):
            a_off = (g // 8) * 512 + (g % 8) * 16
            a0 = attr_v[pl.ds(a_off, 16)]
            a1 = attr_v[pl.ds(a_off + 128, 16)]
            a2 = attr_v[pl.ds(a_off + 256, 16)]
            rowbase = a0 * (D1 * D2 * TS) + a1 * (D2 * TS) + a2 * TS
            leb = g // 8
            le0 = (g % 8) * 16
            for d in range(D):
                v = plsc.load_gather(t_v, [rowbase + d])
                out_v[d // 8, leb, d % 8, pl.ds(le0, 16)] = v

    in_copy(wid, attr_va, in_sa).start()

    def super_body(j, carry):
        c0 = wid + (2 * j) * NW

        def stage(c, attr_v, in_sem, out_v, out_sem,
                  c_next, attr_nv, in_nsem):
            @pl.when(c < n_chunks)
            def _():
                in_copy(c, attr_v, in_sem).wait()

                @pl.when(c_next < n_chunks)
                def _():
                    in_copy(c_next, attr_nv, in_nsem).start()

                @pl.when(c >= 2 * NW)
                def _():
                    out_copy(c - 2 * NW, out_v, out_sem).wait()

                compute(attr_v, out_v)
                out_copy(c, out_v, out_sem).start()

        stage(c0, attr_va, in_sa, out_va, out_sa, c0 + NW, attr_vb, in_sb)
        stage(c0 + NW, attr_vb, in_sb, out_vb, out_sb,
              c0 + 2 * NW, attr_va, in_sa)
        return carry

    lax.fori_loop(0, n_super, super_body, 0)

    last_a = ((n_chunks - 1 - wid) // (2 * NW)) * (2 * NW) + wid

    @pl.when(wid < n_chunks)
    def _():
        out_copy(last_a, out_va, out_sa).wait()

    last_b = ((n_chunks - 1 - wid - NW) // (2 * NW)) * (2 * NW) + wid + NW

    @pl.when(wid + NW < n_chunks)
    def _():
        out_copy(last_b, out_vb, out_sb).wait()


def kernel(edge_attr, W0, W1, W2):
    E = edge_attr.shape[0]
    assert E % CHUNK == 0
    n_chunks = E // CHUNK
    n_iters = (n_chunks + NW - 1) // NW
    n_super = (n_iters + 1) // 2
    EB = E // 128

    attr = jnp.asarray(edge_attr, jnp.int32)
    attr = attr.T.reshape(3, EB, 128).transpose(1, 0, 2)
    attr = jnp.pad(attr, ((0, 0), (0, 1), (0, 0))).reshape(-1)

    mesh = plsc.VectorSubcoreMesh(core_axis_name="c", subcore_axis_name="s")
    body = functools.partial(_sc_body, n_chunks, n_super)
    out4 = pl.kernel(
        body,
        out_type=jax.ShapeDtypeStruct((4, EB, 8, 128), jnp.float32),
        mesh=mesh,
        compiler_params=pltpu.CompilerParams(needs_layout_passes=False),
        scratch_types=[
            pltpu.VMEM((D0 * D,), jnp.float32),
            pltpu.VMEM((D1 * D,), jnp.float32),
            pltpu.VMEM((D2 * D,), jnp.float32),
            pltpu.VMEM((R * TS,), jnp.float32),
            pltpu.VMEM((CB * 512,), jnp.int32),
            pltpu.VMEM((CB * 512,), jnp.int32),
            pltpu.VMEM((4, CB, 8, 128), jnp.float32),
            pltpu.VMEM((4, CB, 8, 128), jnp.float32),
            pltpu.SemaphoreType.DMA,
            pltpu.SemaphoreType.DMA,
            pltpu.SemaphoreType.DMA,
            pltpu.SemaphoreType.DMA,
        ],
    )(attr, W0.reshape(-1), W1.reshape(-1), W2.reshape(-1))
    return out4.transpose(1, 3, 0, 2).reshape(E, D)
